# baseline (speedup 1.0000x reference)
import functools

import jax
import jax.numpy as jnp
from jax.experimental import pallas as pl

_N = 10000
_BR = 400
_EPS = 1e-5


def _stats_body(h_ref, s_ref, q_ref):
    i = pl.program_id(0)
    hb = h_ref[...]
    ps = jnp.sum(hb, axis=0, keepdims=True)
    pq = jnp.sum(hb * hb, axis=0, keepdims=True)

    @pl.when(i == 0)
    def _init():
        s_ref[...] = ps
        q_ref[...] = pq

    @pl.when(i != 0)
    def _acc():
        s_ref[...] = s_ref[...] + ps
        q_ref[...] = q_ref[...] + pq


def _col_stats(h):
    n, c = h.shape
    s, q = pl.pallas_call(
        _stats_body,
        grid=(n // _BR,),
        in_specs=[pl.BlockSpec((_BR, c), lambda i: (i, 0))],
        out_specs=[
            pl.BlockSpec((1, c), lambda i: (0, 0)),
            pl.BlockSpec((1, c), lambda i: (0, 0)),
        ],
        out_shape=[
            jax.ShapeDtypeStruct((1, c), jnp.float32),
            jax.ShapeDtypeStruct((1, c), jnp.float32),
        ],
    )(h)
    mu = s[0] / n
    var = q[0] / n - mu * mu
    return mu, var


def _affmm_body(h_ref, sc_ref, sh_ref, w_ref, o_ref, *, relu):
    hb = h_ref[...] * sc_ref[...] + sh_ref[...]
    if relu:
        hb = jnp.maximum(hb, 0.0)
    o_ref[...] = jnp.dot(hb, w_ref[...], preferred_element_type=jnp.float32)


def _aff_mm(h, scale, shift, w, relu):
    n, k = h.shape
    co = w.shape[1]
    return pl.pallas_call(
        functools.partial(_affmm_body, relu=relu),
        grid=(n // _BR,),
        in_specs=[
            pl.BlockSpec((_BR, k), lambda i: (i, 0)),
            pl.BlockSpec((1, k), lambda i: (0, 0)),
            pl.BlockSpec((1, k), lambda i: (0, 0)),
            pl.BlockSpec((k, co), lambda i: (0, 0)),
        ],
        out_specs=pl.BlockSpec((_BR, co), lambda i: (i, 0)),
        out_shape=jax.ShapeDtypeStruct((n, co), jnp.float32),
    )(h, scale.reshape(1, k), shift.reshape(1, k), w)


def _final_body(h_ref, c4_ref, w5_ref, c5_ref, o_ref):
    hb = jnp.maximum(h_ref[...] + c4_ref[...], 0.0)
    v = jnp.sum(hb * w5_ref[...], axis=1, keepdims=True) + c5_ref[...]
    o_ref[...] = jax.nn.sigmoid(v)


def _final(h, c4, w5, c5):
    n, k = h.shape
    return pl.pallas_call(
        _final_body,
        grid=(n // _BR,),
        in_specs=[
            pl.BlockSpec((_BR, k), lambda i: (i, 0)),
            pl.BlockSpec((1, k), lambda i: (0, 0)),
            pl.BlockSpec((1, k), lambda i: (0, 0)),
            pl.BlockSpec((1, 1), lambda i: (0, 0)),
        ],
        out_specs=pl.BlockSpec((_BR, 1), lambda i: (i, 0)),
        out_shape=jax.ShapeDtypeStruct((n, 1), jnp.float32),
    )(h, c4.reshape(1, k), w5.reshape(1, k), c5.reshape(1, 1))


def kernel(x, edge_index, train_mask, y, g0, b0, W1, c1, g1, be1, W2, c2,
           g2, be2, W3, c3, g3, be3, W4, c4, W5, c5):
    loop = jnp.arange(_N, dtype=edge_index.dtype)
    src = jnp.concatenate([edge_index[0], loop])
    dst = jnp.concatenate([edge_index[1], loop])
    deg = jax.ops.segment_sum(
        jnp.ones_like(src, dtype=jnp.float32), dst, num_segments=_N)
    dinv = jnp.where(deg > 0, deg ** -0.5, 0.0)
    norm = dinv[src] * dinv[dst]

    def agg(t):
        m = jnp.take(t, src, axis=0) * norm[:, None]
        return jax.ops.segment_sum(m, dst, num_segments=_N)

    mu, var = _col_stats(x)
    sc = g0 * jax.lax.rsqrt(var + _EPS)
    sh = b0 - mu * sc
    h = agg(_aff_mm(x, sc, sh, W1, relu=False))

    for (W, g, be) in ((W2, g1, be1), (W3, g2, be2), (W4, g3, be3)):
        mu, var = _col_stats(h)
        sc = g * jax.lax.rsqrt(var + _EPS)
        sh = be - mu * sc
        h = agg(_aff_mm(h, sc, sh, W, relu=True))

    p = _final(h, c4, W5, c5)[:, 0]
    idx = jnp.nonzero(train_mask, size=train_mask.shape[0], fill_value=0)[0]
    return (jnp.take(p, idx, axis=0), jnp.take(y, idx, axis=0))


# speedup vs baseline: 3.0582x; 3.0582x over previous
import functools

import jax
import jax.numpy as jnp
from jax.experimental import pallas as pl

_N = 10000
_BR = 400
_EPS = 1e-5


def _stats_body(h_ref, s_ref, q_ref):
    i = pl.program_id(0)
    hb = h_ref[...]
    ps = jnp.sum(hb, axis=0, keepdims=True)
    pq = jnp.sum(hb * hb, axis=0, keepdims=True)

    @pl.when(i == 0)
    def _init():
        s_ref[...] = ps
        q_ref[...] = pq

    @pl.when(i != 0)
    def _acc():
        s_ref[...] = s_ref[...] + ps
        q_ref[...] = q_ref[...] + pq


def _col_stats(h):
    n, c = h.shape
    s, q = pl.pallas_call(
        _stats_body,
        grid=(n // _BR,),
        in_specs=[pl.BlockSpec((_BR, c), lambda i: (i, 0))],
        out_specs=[
            pl.BlockSpec((1, c), lambda i: (0, 0)),
            pl.BlockSpec((1, c), lambda i: (0, 0)),
        ],
        out_shape=[
            jax.ShapeDtypeStruct((1, c), jnp.float32),
            jax.ShapeDtypeStruct((1, c), jnp.float32),
        ],
    )(h)
    mu = s[0] / n
    var = q[0] / n - mu * mu
    return mu, var


def _affmm_body(h_ref, sc_ref, sh_ref, w_ref, o_ref, *, relu):
    hb = h_ref[...] * sc_ref[...] + sh_ref[...]
    if relu:
        hb = jnp.maximum(hb, 0.0)
    o_ref[...] = jnp.dot(hb, w_ref[...], preferred_element_type=jnp.float32)


def _aff_mm(h, scale, shift, w, relu):
    n, k = h.shape
    co = w.shape[1]
    return pl.pallas_call(
        functools.partial(_affmm_body, relu=relu),
        grid=(n // _BR,),
        in_specs=[
            pl.BlockSpec((_BR, k), lambda i: (i, 0)),
            pl.BlockSpec((1, k), lambda i: (0, 0)),
            pl.BlockSpec((1, k), lambda i: (0, 0)),
            pl.BlockSpec((k, co), lambda i: (0, 0)),
        ],
        out_specs=pl.BlockSpec((_BR, co), lambda i: (i, 0)),
        out_shape=jax.ShapeDtypeStruct((n, co), jnp.float32),
    )(h, scale.reshape(1, k), shift.reshape(1, k), w)


def _mm_body(a_ref, b_ref, o_ref):
    k = pl.program_id(1)
    p = jnp.dot(a_ref[...], b_ref[...], preferred_element_type=jnp.float32)

    @pl.when(k == 0)
    def _init():
        o_ref[...] = p

    @pl.when(k != 0)
    def _acc():
        o_ref[...] = o_ref[...] + p


def _matmul(a, b, bm=1000, bk=1024):
    m, k = a.shape
    n = b.shape[1]
    return pl.pallas_call(
        _mm_body,
        grid=(m // bm, k // bk),
        in_specs=[
            pl.BlockSpec((bm, bk), lambda i, j: (i, j)),
            pl.BlockSpec((bk, n), lambda i, j: (j, 0)),
        ],
        out_specs=pl.BlockSpec((bm, n), lambda i, j: (i, 0)),
        out_shape=jax.ShapeDtypeStruct((m, n), jnp.float32),
    )(a, b)


def _final_body(h_ref, c4_ref, w5_ref, c5_ref, o_ref):
    hb = jnp.maximum(h_ref[...] + c4_ref[...], 0.0)
    v = jnp.sum(hb * w5_ref[...], axis=1, keepdims=True) + c5_ref[...]
    o_ref[...] = jax.nn.sigmoid(v)


def _final(h, c4, w5, c5):
    n, k = h.shape
    return pl.pallas_call(
        _final_body,
        grid=(n // _BR,),
        in_specs=[
            pl.BlockSpec((_BR, k), lambda i: (i, 0)),
            pl.BlockSpec((1, k), lambda i: (0, 0)),
            pl.BlockSpec((1, k), lambda i: (0, 0)),
            pl.BlockSpec((1, 1), lambda i: (0, 0)),
        ],
        out_specs=pl.BlockSpec((_BR, 1), lambda i: (i, 0)),
        out_shape=jax.ShapeDtypeStruct((n, 1), jnp.float32),
    )(h, c4.reshape(1, k), w5.reshape(1, k), c5.reshape(1, 1))


def kernel(x, edge_index, train_mask, y, g0, b0, W1, c1, g1, be1, W2, c2,
           g2, be2, W3, c3, g3, be3, W4, c4, W5, c5):
    loop = jnp.arange(_N, dtype=edge_index.dtype)
    src = jnp.concatenate([edge_index[0], loop])
    dst = jnp.concatenate([edge_index[1], loop])
    deg = jax.ops.segment_sum(
        jnp.ones_like(src, dtype=jnp.float32), dst, num_segments=_N)
    dinv = jnp.where(deg > 0, deg ** -0.5, 0.0)
    norm = dinv[src] * dinv[dst]

    _KP = 10240
    A = (jnp.zeros((_N * _KP,), jnp.float32)
         .at[dst * _KP + src].add(norm)
         .reshape(_N, _KP))

    def _padk(t):
        return jnp.pad(t, ((0, _KP - _N), (0, 0)))

    mu, var = _col_stats(x)
    sc = g0 * jax.lax.rsqrt(var + _EPS)
    sh = b0 - mu * sc
    h = _matmul(A, _padk(_aff_mm(x, sc, sh, W1, relu=False)))

    for (W, g, be) in ((W2, g1, be1), (W3, g2, be2), (W4, g3, be3)):
        mu, var = _col_stats(h)
        sc = g * jax.lax.rsqrt(var + _EPS)
        sh = be - mu * sc
        h = _matmul(A, _padk(_aff_mm(h, sc, sh, W, relu=True)))

    p = _final(h, c4, W5, c5)[:, 0]
    idx = jnp.nonzero(train_mask, size=train_mask.shape[0], fill_value=0)[0]
    return (jnp.take(p, idx, axis=0), jnp.take(y, idx, axis=0))


# speedup vs baseline: 3.1229x; 1.0211x over previous
import functools

import jax
import jax.numpy as jnp
from jax.experimental import pallas as pl

_N = 10000
_BR = 400
_EPS = 1e-5


def _stats_body(h_ref, s_ref, q_ref):
    i = pl.program_id(0)
    hb = h_ref[...]
    ps = jnp.sum(hb, axis=0, keepdims=True)
    pq = jnp.sum(hb * hb, axis=0, keepdims=True)

    @pl.when(i == 0)
    def _init():
        s_ref[...] = ps
        q_ref[...] = pq

    @pl.when(i != 0)
    def _acc():
        s_ref[...] = s_ref[...] + ps
        q_ref[...] = q_ref[...] + pq


def _col_stats(h):
    n, c = h.shape
    s, q = pl.pallas_call(
        _stats_body,
        grid=(n // _BR,),
        in_specs=[pl.BlockSpec((_BR, c), lambda i: (i, 0))],
        out_specs=[
            pl.BlockSpec((1, c), lambda i: (0, 0)),
            pl.BlockSpec((1, c), lambda i: (0, 0)),
        ],
        out_shape=[
            jax.ShapeDtypeStruct((1, c), jnp.float32),
            jax.ShapeDtypeStruct((1, c), jnp.float32),
        ],
    )(h)
    mu = s[0] / n
    var = q[0] / n - mu * mu
    return mu, var


def _affmm_body(h_ref, sc_ref, sh_ref, w_ref, o_ref, *, relu):
    hb = h_ref[...] * sc_ref[...] + sh_ref[...]
    if relu:
        hb = jnp.maximum(hb, 0.0)
    o_ref[...] = jnp.dot(hb, w_ref[...], preferred_element_type=jnp.float32)


def _aff_mm(h, scale, shift, w, relu):
    n, k = h.shape
    co = w.shape[1]
    return pl.pallas_call(
        functools.partial(_affmm_body, relu=relu),
        grid=(n // _BR,),
        in_specs=[
            pl.BlockSpec((_BR, k), lambda i: (i, 0)),
            pl.BlockSpec((1, k), lambda i: (0, 0)),
            pl.BlockSpec((1, k), lambda i: (0, 0)),
            pl.BlockSpec((k, co), lambda i: (0, 0)),
        ],
        out_specs=pl.BlockSpec((_BR, co), lambda i: (i, 0)),
        out_shape=jax.ShapeDtypeStruct((n, co), jnp.float32),
    )(h, scale.reshape(1, k), shift.reshape(1, k), w)


def _mm_body(a_ref, b_ref, o_ref):
    k = pl.program_id(1)
    p = jnp.dot(a_ref[...], b_ref[...], preferred_element_type=jnp.float32)

    @pl.when(k == 0)
    def _init():
        o_ref[...] = p

    @pl.when(k != 0)
    def _acc():
        o_ref[...] = o_ref[...] + p


def _matmul(a, b, bm=1000, bk=1024):
    m, k = a.shape
    n = b.shape[1]
    return pl.pallas_call(
        _mm_body,
        grid=(m // bm, k // bk),
        in_specs=[
            pl.BlockSpec((bm, bk), lambda i, j: (i, j)),
            pl.BlockSpec((bk, n), lambda i, j: (j, 0)),
        ],
        out_specs=pl.BlockSpec((bm, n), lambda i, j: (i, 0)),
        out_shape=jax.ShapeDtypeStruct((m, n), jnp.float32),
    )(a, b)


def _final_body(h_ref, c4_ref, w5_ref, c5_ref, o_ref):
    hb = jnp.maximum(h_ref[...] + c4_ref[...], 0.0)
    v = jnp.sum(hb * w5_ref[...], axis=1, keepdims=True) + c5_ref[...]
    o_ref[...] = jax.nn.sigmoid(v)


def _final(h, c4, w5, c5):
    n, k = h.shape
    return pl.pallas_call(
        _final_body,
        grid=(n // _BR,),
        in_specs=[
            pl.BlockSpec((_BR, k), lambda i: (i, 0)),
            pl.BlockSpec((1, k), lambda i: (0, 0)),
            pl.BlockSpec((1, k), lambda i: (0, 0)),
            pl.BlockSpec((1, 1), lambda i: (0, 0)),
        ],
        out_specs=pl.BlockSpec((_BR, 1), lambda i: (i, 0)),
        out_shape=jax.ShapeDtypeStruct((n, 1), jnp.float32),
    )(h, c4.reshape(1, k), w5.reshape(1, k), c5.reshape(1, 1))


def kernel(x, edge_index, train_mask, y, g0, b0, W1, c1, g1, be1, W2, c2,
           g2, be2, W3, c3, g3, be3, W4, c4, W5, c5):
    loop = jnp.arange(_N, dtype=edge_index.dtype)
    src = jnp.concatenate([edge_index[0], loop])
    dst = jnp.concatenate([edge_index[1], loop])
    deg = jax.ops.segment_sum(
        jnp.ones_like(src, dtype=jnp.float32), dst, num_segments=_N)
    dinv = jnp.where(deg > 0, deg ** -0.5, 0.0)
    norm = dinv[src] * dinv[dst]

    _KP = 10240
    A = (jnp.zeros((_N * _KP,), jnp.float32)
         .at[dst * _KP + src].add(norm)
         .reshape(_N, _KP)
         .astype(jnp.bfloat16))

    def _padk(t):
        return jnp.pad(t, ((0, _KP - _N), (0, 0))).astype(jnp.bfloat16)

    mu, var = _col_stats(x)
    sc = g0 * jax.lax.rsqrt(var + _EPS)
    sh = b0 - mu * sc
    h = _matmul(A, _padk(_aff_mm(x, sc, sh, W1, relu=False)))

    for (W, g, be) in ((W2, g1, be1), (W3, g2, be2), (W4, g3, be3)):
        mu, var = _col_stats(h)
        sc = g * jax.lax.rsqrt(var + _EPS)
        sh = be - mu * sc
        h = _matmul(A, _padk(_aff_mm(h, sc, sh, W, relu=True)))

    p = _final(h, c4, W5, c5)[:, 0]
    idx = jnp.nonzero(train_mask, size=train_mask.shape[0], fill_value=0)[0]
    return (jnp.take(p, idx, axis=0), jnp.take(y, idx, axis=0))
